# baseline (speedup 1.0000x reference)
import functools
import math

import jax
import jax.numpy as jnp
from jax import lax
from jax.experimental import pallas as pl
from jax.experimental.pallas import tpu as pltpu
from jax.experimental.pallas import tpu_sc as plsc

_B, _N, _NCLS = 4, 20000, 5
_ROWS = _B * _N
_MPPX = 0.8
_MPPY = 0.8
_THRESH = 0.5

_WPB = 8
_CHUNK = 2512
_GROUPS = _CHUNK // 16

_HALF_PI = math.pi / 2.0
_QTR_PI = math.pi / 4.0
_TAN_PI_8 = 0.4142135623730951


def _atan2(yv, xv):
    ax = jnp.abs(xv)
    ay = jnp.abs(yv)
    hi = jnp.maximum(ax, ay)
    lo = jnp.minimum(ax, ay)
    a = lo / jnp.maximum(hi, 1e-30)
    big = a > _TAN_PI_8
    t = jnp.where(big, (a - 1.0) / (a + 1.0), a)
    s = t * t
    p = (((8.05374449538e-2 * s - 1.38776856032e-1) * s + 1.99777106478e-1)
         * s - 3.33329491539e-1) * s * t + t
    r = jnp.where(big, p + _QTR_PI, p)
    r = jnp.where(ay > ax, _HALF_PI - r, r)
    r = jnp.where(xv < 0.0, math.pi - r, r)
    return jnp.where(yv < 0.0, -r, r)


def _decode_row(cls_c, ctr_c, box_c, dir_c, scr_c):
    r = [jnp.maximum(c, 0.0) for c in cls_c]
    ssum = r[0] + r[1] + r[2] + r[3] + r[4] + 5.0
    conf = [(rj + 1.0) / ssum for rj in r]
    m14 = jnp.maximum(jnp.maximum(conf[1], conf[2]),
                      jnp.maximum(conf[3], conf[4]))
    maskb = m14 > _THRESH
    scr = jnp.maximum(m14, conf[0])

    x = ctr_c[0] + box_c[0] * _MPPX
    y = ctr_c[1] + box_c[1] * _MPPY
    z = box_c[2]
    l = jnp.exp(jnp.maximum(jnp.minimum(box_c[3], 4.0), -4.0))
    w = jnp.exp(jnp.maximum(jnp.minimum(box_c[4], 4.0), -4.0))
    h = jnp.exp(jnp.maximum(jnp.minimum(box_c[5], 4.0), -4.0))

    best = scr_c[0]
    binf = jnp.zeros_like(best)
    sc0 = dir_c[0]
    sc1 = dir_c[1]
    for j in (1, 2, 3):
        gt = scr_c[j] > best
        best = jnp.where(gt, scr_c[j], best)
        binf = jnp.where(gt, float(j), binf)
        sc0 = jnp.where(gt, dir_c[2 * j], sc0)
        sc1 = jnp.where(gt, dir_c[2 * j + 1], sc1)
    ry = _atan2(sc0, sc1) + binf * _HALF_PI
    esum = (jnp.exp(scr_c[0] - best) + jnp.exp(scr_c[1] - best)
            + jnp.exp(scr_c[2] - best) + jnp.exp(scr_c[3] - best))
    dir_scr = 1.0 / esum

    mf = jnp.where(maskb, 1.0, 0.0)
    return [x * mf, y * mf, z * mf, l * mf, w * mf, h * mf,
            ry * mf, dir_scr * mf, scr * mf]


def _sc_decode_body(cls_h, ctr_h, box_h, dir_h, scr_h, out_hbm,
                    in_v, out_v, sem):
    wid = lax.axis_index("s") * 2 + lax.axis_index("c")
    b = wid // _WPB
    n0 = jnp.minimum((wid % _WPB) * _CHUNK, _N - _CHUNK)
    r0 = b * _N + n0

    plan = [
        (cls_h, 5, 0),
        (ctr_h, 2, 5 * _CHUNK),
        (box_h, 6, 7 * _CHUNK),
        (dir_h, 8, 13 * _CHUNK),
        (scr_h, 4, 21 * _CHUNK),
    ]
    copies = [
        pltpu.async_copy(ref.at[pl.ds(r0 * c, _CHUNK * c)],
                         in_v.at[pl.ds(off, _CHUNK * c)], sem)
        for ref, c, off in plan
    ]
    for c in copies:
        c.wait()

    iota = lax.iota(jnp.int32, 16)
    striota = {c: iota * c for c in (2, 4, 5, 6, 8, 9)}

    @plsc.parallel_loop(0, _GROUPS, unroll=4)
    def body(g):
        o = g * 16

        def ld(nch, off):
            return [plsc.load_gather(in_v, [striota[nch] + (off + o * nch + c)])
                    for c in range(nch)]

        cls_c = ld(5, 0)
        ctr_c = ld(2, 5 * _CHUNK)
        box_c = ld(6, 7 * _CHUNK)
        dir_c = ld(8, 13 * _CHUNK)
        scr_c = ld(4, 21 * _CHUNK)

        outs = _decode_row(cls_c, ctr_c, box_c, dir_c, scr_c)
        for ch, val in enumerate(outs):
            plsc.store_scatter(out_v, [striota[9] + (o * 9 + ch)], val)

    pltpu.async_copy(out_v.at[pl.ds(0, _CHUNK * 9)],
                     out_hbm.at[pl.ds(r0 * 9, _CHUNK * 9)], sem).wait()


@functools.lru_cache(maxsize=1)
def _sc_decode():
    return pl.kernel(
        _sc_decode_body,
        mesh=plsc.VectorSubcoreMesh(core_axis_name="c", subcore_axis_name="s"),
        compiler_params=pltpu.CompilerParams(needs_layout_passes=False),
        out_type=jax.ShapeDtypeStruct((9 * _ROWS,), jnp.float32),
        scratch_types=[
            pltpu.VMEM((_CHUNK * 25 + 48,), jnp.float32),
            pltpu.VMEM((_CHUNK * 9 + 48,), jnp.float32),
            pltpu.SemaphoreType.DMA,
        ],
    )


def kernel(cls, ctr, reg_box, reg_dir, reg_scr):
    out = _sc_decode()(
        cls.reshape(-1),
        ctr.reshape(-1),
        reg_box.reshape(-1),
        reg_dir.reshape(-1),
        reg_scr.reshape(-1),
    )
    return out.reshape(_ROWS, 9)


# speedup vs baseline: 7.9306x; 7.9306x over previous
import functools
import math

import jax
import jax.numpy as jnp
from jax import lax
from jax.experimental import pallas as pl
from jax.experimental.pallas import tpu as pltpu
from jax.experimental.pallas import tpu_sc as plsc

_B, _N, _NCLS = 4, 20000, 5
_ROWS = _B * _N
_MPPX = 0.8
_MPPY = 0.8
_THRESH = 0.5

_WPB = 8
_CHUNK = 2512
_GROUPS = _CHUNK // 16

_HALF_PI = math.pi / 2.0
_QTR_PI = math.pi / 4.0
_TAN_PI_8 = 0.4142135623730951


def _atan2(yv, xv):
    ax = jnp.abs(xv)
    ay = jnp.abs(yv)
    hi = jnp.maximum(ax, ay)
    lo = jnp.minimum(ax, ay)
    a = lo / jnp.maximum(hi, 1e-30)
    big = a > _TAN_PI_8
    t = jnp.where(big, (a - 1.0) / (a + 1.0), a)
    s = t * t
    p = (((8.05374449538e-2 * s - 1.38776856032e-1) * s + 1.99777106478e-1)
         * s - 3.33329491539e-1) * s * t + t
    r = jnp.where(big, p + _QTR_PI, p)
    r = jnp.where(ay > ax, _HALF_PI - r, r)
    r = jnp.where(xv < 0.0, math.pi - r, r)
    return jnp.where(yv < 0.0, -r, r)


def _decode_row(cls_c, ctr_c, box_c, dir_c, scr_c):
    r = [jnp.maximum(c, 0.0) for c in cls_c]
    ssum = (r[0] + r[1]) + (r[2] + r[3]) + (r[4] + 5.0)
    rinv = 1.0 / ssum
    amax14 = jnp.maximum(jnp.maximum(r[1], r[2]), jnp.maximum(r[3], r[4]))
    m14 = (amax14 + 1.0) * rinv
    maskb = m14 > _THRESH
    scr = jnp.maximum(m14, (r[0] + 1.0) * rinv)

    x = ctr_c[0] + box_c[0] * _MPPX
    y = ctr_c[1] + box_c[1] * _MPPY
    z = box_c[2]
    l = jnp.exp(jnp.maximum(jnp.minimum(box_c[3], 4.0), -4.0))
    w = jnp.exp(jnp.maximum(jnp.minimum(box_c[4], 4.0), -4.0))
    h = jnp.exp(jnp.maximum(jnp.minimum(box_c[5], 4.0), -4.0))

    best = scr_c[0]
    binf = jnp.zeros_like(best)
    sc0 = dir_c[0]
    sc1 = dir_c[1]
    for j in (1, 2, 3):
        gt = scr_c[j] > best
        best = jnp.where(gt, scr_c[j], best)
        binf = jnp.where(gt, float(j), binf)
        sc0 = jnp.where(gt, dir_c[2 * j], sc0)
        sc1 = jnp.where(gt, dir_c[2 * j + 1], sc1)
    ry = _atan2(sc0, sc1) + binf * _HALF_PI
    esum = (jnp.exp(scr_c[0] - best) + jnp.exp(scr_c[1] - best)
            + jnp.exp(scr_c[2] - best) + jnp.exp(scr_c[3] - best))
    dir_scr = 1.0 / esum

    mf = jnp.where(maskb, 1.0, 0.0)
    return [x * mf, y * mf, z * mf, l * mf, w * mf, h * mf,
            ry * mf, dir_scr * mf, scr * mf]


def _sc_decode_body(cls_h, ctr_h, box_h, dir_h, scr_h, out_hbm,
                    in_v, out_v, sem):
    wid = lax.axis_index("s") * 2 + lax.axis_index("c")
    b = wid // _WPB
    n0 = jnp.minimum((wid % _WPB) * _CHUNK, _N - _CHUNK)

    def src_off(kind, nch, ch):
        if kind == "cbn":
            return (ch * _B + b) * _N + n0
        return (b * nch + ch) * _N + n0

    plan = [
        (cls_h, "cbn", 5),
        (ctr_h, "bcn", 2),
        (box_h, "cbn", 6),
        (dir_h, "bcn", 8),
        (scr_h, "bcn", 4),
    ]
    copies = []
    slot = 0
    for ref, kind, nch in plan:
        for ch in range(nch):
            copies.append(pltpu.async_copy(
                ref.at[pl.ds(src_off(kind, nch, ch), _CHUNK)],
                in_v.at[pl.ds(slot * _CHUNK, _CHUNK)], sem))
            slot += 1
    for c in copies:
        c.wait()

    @plsc.parallel_loop(0, _GROUPS, unroll=4)
    def body(g):
        o = g * 16

        def ld(slot_base, count):
            return [in_v[pl.ds((slot_base + j) * _CHUNK + o, 16)]
                    for j in range(count)]

        cls_c = ld(0, 5)
        ctr_c = ld(5, 2)
        box_c = ld(7, 6)
        dir_c = ld(13, 8)
        scr_c = ld(21, 4)

        outs = _decode_row(cls_c, ctr_c, box_c, dir_c, scr_c)
        for ch, val in enumerate(outs):
            out_v[pl.ds(ch * _CHUNK + o, 16)] = val

    out_copies = [
        pltpu.async_copy(out_v.at[pl.ds(ch * _CHUNK, _CHUNK)],
                         out_hbm.at[pl.ds((ch * _B + b) * _N + n0, _CHUNK)],
                         sem)
        for ch in range(9)
    ]
    for c in out_copies:
        c.wait()


@functools.lru_cache(maxsize=1)
def _sc_decode():
    return pl.kernel(
        _sc_decode_body,
        mesh=plsc.VectorSubcoreMesh(core_axis_name="c", subcore_axis_name="s"),
        out_type=jax.ShapeDtypeStruct((9 * _ROWS,), jnp.float32),
        scratch_types=[
            pltpu.VMEM((_CHUNK * 25,), jnp.float32),
            pltpu.VMEM((_CHUNK * 9,), jnp.float32),
            pltpu.SemaphoreType.DMA,
        ],
    )


def kernel(cls, ctr, reg_box, reg_dir, reg_scr):
    out = _sc_decode()(
        cls.transpose(2, 0, 1).reshape(-1),
        ctr.transpose(0, 2, 1).reshape(-1),
        reg_box.transpose(2, 0, 1).reshape(-1),
        reg_dir.transpose(0, 2, 1).reshape(-1),
        reg_scr.transpose(0, 2, 1).reshape(-1),
    )
    return out.reshape(9, _ROWS).T
